# baseline (speedup 1.0000x reference)
import jax
import jax.numpy as jnp
from jax import lax
from jax.experimental import pallas as pl
from jax.experimental.pallas import tpu as pltpu

_DN = (((1,), (0,)), ((), ()))


def _pick(n, cap):
    for d in range(min(cap, n), 0, -1):
        if n % d == 0 and d % 8 == 0:
            return d
    return n


def _hdot(a, b):
    a_hi = a.astype(jnp.bfloat16)
    a_lo = (a - a_hi.astype(jnp.float32)).astype(jnp.bfloat16)
    b_hi = b.astype(jnp.bfloat16)
    b_lo = (b - b_hi.astype(jnp.float32)).astype(jnp.bfloat16)
    p = jnp.dot(a_hi, b_hi, preferred_element_type=jnp.float32)
    p += jnp.dot(a_hi, b_lo, preferred_element_type=jnp.float32)
    p += jnp.dot(a_lo, b_hi, preferred_element_type=jnp.float32)
    return p


def _make_dinv_cast_kernel(bi):
    def _dinv_cast_kernel(a_ref, o_ref, oc_ref, b16t_ref):
        i = pl.program_id(0)
        a = a_ref[...]
        rows = lax.broadcasted_iota(jnp.int32, a.shape, 0) + i * bi
        cols = lax.broadcasted_iota(jnp.int32, a.shape, 1)
        b = a + 2.0 * (rows == cols).astype(jnp.float32)
        b16t_ref[0] = jnp.transpose(b.astype(jnp.float8_e4m3fn), (1, 0))
        part = jnp.sum(b, axis=0, keepdims=True)

        @pl.when(i == 0)
        def _():
            o_ref[...] = part

        @pl.when(i > 0)
        def _():
            o_ref[...] += part

        @pl.when(i == pl.num_programs(0) - 1)
        def _():
            d = lax.rsqrt(o_ref[...])
            o_ref[...] = d
            oc_ref[...] = jnp.transpose(d, (1, 0))

    return _dinv_cast_kernel


def _accumulate_chunks(i, at_ref, g16, o_ref, n, bc):
    for c in range(n // bc):
        sl = pl.ds(c * bc, bc)
        p = lax.dot_general(at_ref[0, sl, :].astype(jnp.bfloat16), g16, _DN,
                            preferred_element_type=jnp.float32)
        o_ref[sl, :] = jnp.where(i == 0, p, o_ref[sl, :] + p)


def _make_agg1_kernel(n, bc):
    def _agg1_kernel(a_ref, w1_ref, di_ref, o_ref):
        i = pl.program_id(0)
        g16 = (w1_ref[...] * di_ref[...]).astype(jnp.bfloat16)
        _accumulate_chunks(i, a_ref, g16, o_ref, n, bc)

    return _agg1_kernel


def _accumulate_chunks_fp8(i, at_ref, hi8, lo8, o_ref, n, bc):
    for c in range(n // bc):
        sl = pl.ds(c * bc, bc)
        a8 = at_ref[0, sl, :]
        p = lax.dot_general(a8, hi8, _DN, preferred_element_type=jnp.float32)
        p = p * (1.0 / 128.0)
        p += lax.dot_general(a8, lo8, _DN,
                             preferred_element_type=jnp.float32) * (1.0 / (128.0 * 512.0))
        o_ref[sl, :] = jnp.where(i == 0, p, o_ref[sl, :] + p)


def _make_agg2_epi_kernel(n, bc, bi, bj, ni):
    def _agg2_epi_kernel(a_ref, agg1_ref, di_ref, dj_ref, b1_ref, w2_ref,
                         wr_ref, b2_ref, br_ref, wf1_ref, bf1_ref, wf2_ref,
                         bf2_ref, x_ref, A_ref, xr_s, agg2_s):
        k = pl.program_id(0)

        @pl.when(k < ni)
        def _():
            di = di_ref[...]
            x1 = jnp.maximum(agg1_ref[...] * di + b1_ref[...], 0.0)
            h2 = _hdot(x1, w2_ref[...])
            xr_s[pl.ds(k * bi, bi), :] = _hdot(x1, wr_ref[...])
            g2s = h2 * di * 128.0
            hi8 = g2s.astype(jnp.float8_e4m3fn)
            lo8 = ((g2s - hi8.astype(jnp.float32)) * 512.0).astype(
                jnp.float8_e4m3fn)
            _accumulate_chunks_fp8(k, a_ref, hi8, lo8, agg2_s, n, bc)

        @pl.when(k >= ni)
        def _():
            sl = pl.ds((k - ni) * bj, bj)
            x2 = jnp.maximum(agg2_s[sl, :] * dj_ref[...] + b2_ref[...], 0.0)
            x = x2 + xr_s[sl, :] + br_ref[...]
            x_ref[...] = x
            t = jnp.maximum(_hdot(x, wf1_ref[...]) + bf1_ref[...], 0.0)
            A_ref[...] = _hdot(t, wf2_ref[...]) + bf2_ref[...]

    return _agg2_epi_kernel


def kernel(adj, x_eye, W1, b1, W2, b2, Wr, br, Wf1, bf1, Wf2, bf2):
    del x_eye
    n = adj.shape[0]
    f1 = W1.shape[1]
    f2 = W2.shape[1]
    fh = Wf1.shape[1]

    bi = _pick(n, 400)
    bc = _pick(n, 2000)
    ni = n // bi
    dinv, dcol, b16 = pl.pallas_call(
        _make_dinv_cast_kernel(bi),
        grid=(ni,),
        in_specs=[pl.BlockSpec((bi, n), lambda i: (i, 0))],
        out_specs=[
            pl.BlockSpec((1, n), lambda i: (0, 0)),
            pl.BlockSpec((n, 1), lambda i: (0, 0)),
            pl.BlockSpec((1, n, bi), lambda i: (i, 0, 0)),
        ],
        out_shape=[
            jax.ShapeDtypeStruct((1, n), jnp.float32),
            jax.ShapeDtypeStruct((n, 1), jnp.float32),
            jax.ShapeDtypeStruct((ni, n, bi), jnp.float8_e4m3fn),
        ],
        compiler_params=pltpu.CompilerParams(
            dimension_semantics=("arbitrary",)),
    )(adj)

    b1r = jnp.reshape(b1, (1, f1))
    b2r = jnp.reshape(b2, (1, f2))
    brr = jnp.reshape(br, (1, f2))
    bf1r = jnp.reshape(bf1, (1, fh))
    bf2r = jnp.reshape(bf2, (1, 1))

    agg1 = pl.pallas_call(
        _make_agg1_kernel(n, bc),
        grid=(ni,),
        in_specs=[
            pl.BlockSpec((1, n, bi), lambda i: (i, 0, 0)),
            pl.BlockSpec((bi, f1), lambda i: (i, 0)),
            pl.BlockSpec((bi, 1), lambda i: (i, 0)),
        ],
        out_specs=pl.BlockSpec((n, f1), lambda i: (0, 0)),
        out_shape=jax.ShapeDtypeStruct((n, f1), jnp.float32),
        compiler_params=pltpu.CompilerParams(
            dimension_semantics=("arbitrary",)),
    )(b16, W1, dcol)

    bj = _pick(n, 2000)
    nj = n // bj
    x, A = pl.pallas_call(
        _make_agg2_epi_kernel(n, bc, bi, bj, ni),
        grid=(ni + nj,),
        in_specs=[
            pl.BlockSpec((1, n, bi),
                         lambda k: (jnp.minimum(k, ni - 1), 0, 0)),
            pl.BlockSpec((bi, f1),
                         lambda k: (jnp.minimum(k, ni - 1), 0)),
            pl.BlockSpec((bi, 1),
                         lambda k: (jnp.minimum(k, ni - 1), 0)),
            pl.BlockSpec((bj, 1),
                         lambda k: (jnp.maximum(k - ni, 0), 0)),
            pl.BlockSpec((1, f1), lambda k: (0, 0)),
            pl.BlockSpec((f1, f2), lambda k: (0, 0)),
            pl.BlockSpec((f1, f2), lambda k: (0, 0)),
            pl.BlockSpec((1, f2), lambda k: (0, 0)),
            pl.BlockSpec((1, f2), lambda k: (0, 0)),
            pl.BlockSpec((f2, fh), lambda k: (0, 0)),
            pl.BlockSpec((1, fh), lambda k: (0, 0)),
            pl.BlockSpec((fh, 1), lambda k: (0, 0)),
            pl.BlockSpec((1, 1), lambda k: (0, 0)),
        ],
        out_specs=[
            pl.BlockSpec((bj, f2), lambda k: (jnp.maximum(k - ni, 0), 0)),
            pl.BlockSpec((bj, 1), lambda k: (jnp.maximum(k - ni, 0), 0)),
        ],
        out_shape=[
            jax.ShapeDtypeStruct((n, f2), jnp.float32),
            jax.ShapeDtypeStruct((n, 1), jnp.float32),
        ],
        scratch_shapes=[
            pltpu.VMEM((n, f2), jnp.float32),
            pltpu.VMEM((n, f2), jnp.float32),
        ],
        compiler_params=pltpu.CompilerParams(
            dimension_semantics=("arbitrary",)),
    )(b16, agg1, dcol, dcol, b1r, W2, Wr, b2r, brr, Wf1, bf1r, Wf2, bf2r)

    return (x, A)


# speedup vs baseline: 1.0083x; 1.0083x over previous
import jax
import jax.numpy as jnp
from jax import lax
from jax.experimental import pallas as pl
from jax.experimental.pallas import tpu as pltpu

_DN = (((1,), (0,)), ((), ()))


def _pick(n, cap):
    for d in range(min(cap, n), 0, -1):
        if n % d == 0 and d % 8 == 0:
            return d
    return n


def _hdot(a, b):
    a_hi = a.astype(jnp.bfloat16)
    a_lo = (a - a_hi.astype(jnp.float32)).astype(jnp.bfloat16)
    b_hi = b.astype(jnp.bfloat16)
    b_lo = (b - b_hi.astype(jnp.float32)).astype(jnp.bfloat16)
    p = jnp.dot(a_hi, b_hi, preferred_element_type=jnp.float32)
    p += jnp.dot(a_hi, b_lo, preferred_element_type=jnp.float32)
    p += jnp.dot(a_lo, b_hi, preferred_element_type=jnp.float32)
    return p


def _make_dinv_cast_kernel(bi):
    def _dinv_cast_kernel(a_ref, o_ref, oc_ref, b16t_ref):
        i = pl.program_id(0)
        a = a_ref[...]
        rows = lax.broadcasted_iota(jnp.int32, a.shape, 0) + i * bi
        cols = lax.broadcasted_iota(jnp.int32, a.shape, 1)
        b = a + 2.0 * (rows == cols).astype(jnp.float32)
        b16t_ref[0] = jnp.transpose(b.astype(jnp.float8_e4m3fn), (1, 0))
        part = jnp.sum(b, axis=0, keepdims=True)

        @pl.when(i == 0)
        def _():
            o_ref[...] = part

        @pl.when(i > 0)
        def _():
            o_ref[...] += part

        @pl.when(i == pl.num_programs(0) - 1)
        def _():
            d = lax.rsqrt(o_ref[...])
            o_ref[...] = d
            oc_ref[...] = jnp.transpose(d, (1, 0))

    return _dinv_cast_kernel


def _accumulate_chunks(i, at_ref, g16, o_ref, n, bc):
    for c in range(n // bc):
        sl = pl.ds(c * bc, bc)
        p = lax.dot_general(at_ref[0, sl, :].astype(jnp.bfloat16), g16, _DN,
                            preferred_element_type=jnp.float32)
        o_ref[sl, :] = jnp.where(i == 0, p, o_ref[sl, :] + p)


def _make_agg1_kernel(n, bc):
    def _agg1_kernel(a_ref, w1_ref, di_ref, o_ref):
        i = pl.program_id(0)
        g16 = (w1_ref[...] * di_ref[...]).astype(jnp.bfloat16)
        _accumulate_chunks(i, a_ref, g16, o_ref, n, bc)

    return _agg1_kernel


def _make_agg2_epi_kernel(n, bc, bi, bj, ni):
    def _agg2_epi_kernel(a_ref, agg1_ref, di_ref, dj_ref, b1_ref, w2_ref,
                         wr_ref, b2_ref, br_ref, wf1_ref, bf1_ref, wf2_ref,
                         bf2_ref, x_ref, A_ref, xr_s, agg2_s):
        k = pl.program_id(0)

        @pl.when(k < ni)
        def _():
            di = di_ref[...]
            x1 = jnp.maximum(agg1_ref[...] * di + b1_ref[...], 0.0)
            h2 = _hdot(x1, w2_ref[...])
            xr_s[pl.ds(k * bi, bi), :] = _hdot(x1, wr_ref[...])
            g16 = (h2 * di).astype(jnp.bfloat16)
            _accumulate_chunks(k, a_ref, g16, agg2_s, n, bc)

        @pl.when(k >= ni)
        def _():
            sl = pl.ds((k - ni) * bj, bj)
            x2 = jnp.maximum(agg2_s[sl, :] * dj_ref[...] + b2_ref[...], 0.0)
            x = x2 + xr_s[sl, :] + br_ref[...]
            x_ref[...] = x
            t = jnp.maximum(_hdot(x, wf1_ref[...]) + bf1_ref[...], 0.0)
            A_ref[...] = _hdot(t, wf2_ref[...]) + bf2_ref[...]

    return _agg2_epi_kernel


def kernel(adj, x_eye, W1, b1, W2, b2, Wr, br, Wf1, bf1, Wf2, bf2):
    del x_eye
    n = adj.shape[0]
    f1 = W1.shape[1]
    f2 = W2.shape[1]
    fh = Wf1.shape[1]

    bi = _pick(n, 400)
    bc = _pick(n, 2000)
    ni = n // bi
    dinv, dcol, b16 = pl.pallas_call(
        _make_dinv_cast_kernel(bi),
        grid=(ni,),
        in_specs=[pl.BlockSpec((bi, n), lambda i: (i, 0))],
        out_specs=[
            pl.BlockSpec((1, n), lambda i: (0, 0)),
            pl.BlockSpec((n, 1), lambda i: (0, 0)),
            pl.BlockSpec((1, n, bi), lambda i: (i, 0, 0)),
        ],
        out_shape=[
            jax.ShapeDtypeStruct((1, n), jnp.float32),
            jax.ShapeDtypeStruct((n, 1), jnp.float32),
            jax.ShapeDtypeStruct((ni, n, bi), jnp.float8_e4m3fn),
        ],
        compiler_params=pltpu.CompilerParams(
            dimension_semantics=("arbitrary",)),
    )(adj)

    b1r = jnp.reshape(b1, (1, f1))
    b2r = jnp.reshape(b2, (1, f2))
    brr = jnp.reshape(br, (1, f2))
    bf1r = jnp.reshape(bf1, (1, fh))
    bf2r = jnp.reshape(bf2, (1, 1))

    agg1 = pl.pallas_call(
        _make_agg1_kernel(n, bc),
        grid=(ni,),
        in_specs=[
            pl.BlockSpec((1, n, bi), lambda i: (i, 0, 0)),
            pl.BlockSpec((bi, f1), lambda i: (i, 0)),
            pl.BlockSpec((bi, 1), lambda i: (i, 0)),
        ],
        out_specs=pl.BlockSpec((n, f1), lambda i: (0, 0)),
        out_shape=jax.ShapeDtypeStruct((n, f1), jnp.float32),
        compiler_params=pltpu.CompilerParams(
            dimension_semantics=("arbitrary",)),
    )(b16, W1, dcol)

    bj = _pick(n, 2000)
    nj = n // bj
    x, A = pl.pallas_call(
        _make_agg2_epi_kernel(n, bc, bi, bj, ni),
        grid=(ni + nj,),
        in_specs=[
            pl.BlockSpec((1, n, bi),
                         lambda k: (jnp.minimum(k, ni - 1), 0, 0)),
            pl.BlockSpec((bi, f1),
                         lambda k: (jnp.minimum(k, ni - 1), 0)),
            pl.BlockSpec((bi, 1),
                         lambda k: (jnp.minimum(k, ni - 1), 0)),
            pl.BlockSpec((bj, 1),
                         lambda k: (jnp.maximum(k - ni, 0), 0)),
            pl.BlockSpec((1, f1), lambda k: (0, 0)),
            pl.BlockSpec((f1, f2), lambda k: (0, 0)),
            pl.BlockSpec((f1, f2), lambda k: (0, 0)),
            pl.BlockSpec((1, f2), lambda k: (0, 0)),
            pl.BlockSpec((1, f2), lambda k: (0, 0)),
            pl.BlockSpec((f2, fh), lambda k: (0, 0)),
            pl.BlockSpec((1, fh), lambda k: (0, 0)),
            pl.BlockSpec((fh, 1), lambda k: (0, 0)),
            pl.BlockSpec((1, 1), lambda k: (0, 0)),
        ],
        out_specs=[
            pl.BlockSpec((bj, f2), lambda k: (jnp.maximum(k - ni, 0), 0)),
            pl.BlockSpec((bj, 1), lambda k: (jnp.maximum(k - ni, 0), 0)),
        ],
        out_shape=[
            jax.ShapeDtypeStruct((n, f2), jnp.float32),
            jax.ShapeDtypeStruct((n, 1), jnp.float32),
        ],
        scratch_shapes=[
            pltpu.VMEM((n, f2), jnp.float32),
            pltpu.VMEM((n, f2), jnp.float32),
        ],
        compiler_params=pltpu.CompilerParams(
            dimension_semantics=("arbitrary",)),
    )(b16, agg1, dcol, dcol, b1r, W2, Wr, b2r, brr, Wf1, bf1r, Wf2, bf2r)

    return (x, A)
